# baseline (speedup 1.0000x reference)
import jax
import jax.numpy as jnp
from jax import lax
from jax.experimental import pallas as pl
from jax.experimental.pallas import tpu as pltpu

_N = 16384
_NCHUNK = 4
_CH = _N // _NCHUNK


def _scale(xv, v0, v1, v2):
    s = lax.broadcasted_iota(jnp.int32, (2, 1), 0)
    a = jnp.where(s == 0, v0, v1)
    top = xv[0:2, :] * a
    mid = xv[3:4, :] * v2
    bot = jnp.zeros((1, _CH), jnp.float32)
    return jnp.concatenate([top, mid, bot], axis=0)


def _body(vals_ref, x_hbm, o_hbm, xs_ref, os_ref, in_sems, out_sems):
    v0 = vals_ref[0]
    v1 = vals_ref[1]
    v2 = vals_ref[2]

    def in_copy(i, slot):
        return pltpu.make_async_copy(
            x_hbm.at[:, pl.ds(i * _CH, _CH)], xs_ref.at[slot], in_sems.at[slot]
        )

    def out_copy(i, slot):
        return pltpu.make_async_copy(
            os_ref.at[slot], o_hbm.at[:, pl.ds(i * _CH, _CH)], out_sems.at[slot]
        )

    in_copy(0, 0).start()
    in_copy(1, 1).start()
    for i in range(_NCHUNK):
        slot = i % 2
        in_copy(i, slot).wait()
        if i >= 2:
            out_copy(i - 2, slot).wait()
        os_ref[slot] = _scale(xs_ref[slot], v0, v1, v2)
        out_copy(i, slot).start()
        if i + 2 < _NCHUNK:
            in_copy(i + 2, slot).start()
    out_copy(_NCHUNK - 2, 0).wait()
    out_copy(_NCHUNK - 1, 1).wait()


@jax.jit
def kernel(x, values):
    out_t = pl.pallas_call(
        _body,
        out_shape=jax.ShapeDtypeStruct((4, _N), jnp.float32),
        in_specs=[
            pl.BlockSpec(memory_space=pltpu.SMEM),
            pl.BlockSpec(memory_space=pl.ANY),
        ],
        out_specs=pl.BlockSpec(memory_space=pl.ANY),
        scratch_shapes=[
            pltpu.VMEM((2, 4, _CH), jnp.float32),
            pltpu.VMEM((2, 4, _CH), jnp.float32),
            pltpu.SemaphoreType.DMA((2,)),
            pltpu.SemaphoreType.DMA((2,)),
        ],
    )(values, x.T)
    return out_t.T


# speedup vs baseline: 1.8523x; 1.8523x over previous
import jax
import jax.numpy as jnp
from jax import lax
from jax.experimental import pallas as pl
from jax.experimental.pallas import tpu as pltpu

_N = 16384


def _body(vals_ref, x_ref, o_ref):
    v0 = vals_ref[0]
    v1 = vals_ref[1]
    v2 = vals_ref[2]
    s = lax.broadcasted_iota(jnp.int32, (2, 1), 0)
    a = jnp.where(s == 0, v0, v1)
    o_ref[0:2, :] = x_ref[0:2, :] * a
    o_ref[2:3, :] = x_ref[3:4, :] * v2
    o_ref[3:4, :] = jnp.zeros((1, _N), jnp.float32)


@jax.jit
def kernel(x, values):
    out_t = pl.pallas_call(
        _body,
        out_shape=jax.ShapeDtypeStruct((4, _N), jnp.float32),
        in_specs=[
            pl.BlockSpec(memory_space=pltpu.SMEM),
            pl.BlockSpec(memory_space=pltpu.VMEM),
        ],
        out_specs=pl.BlockSpec(memory_space=pltpu.VMEM),
    )(values, x.T)
    return out_t.T
